# Initial kernel scaffold; baseline (speedup 1.0000x reference)
import jax
import jax.numpy as jnp
from jax.experimental import pallas as pl


def kernel(x, Wg, bg, W1, b1, W2, b2):
    raise NotImplementedError("write your pallas kernel here")


# speedup vs baseline: 3.1105x; 3.1105x over previous
import functools

import jax
import jax.numpy as jnp
from jax.experimental import pallas as pl
from jax.experimental.pallas import tpu as pltpu

N_TOK = 2048
D = 768
E = 8
NEG_INF = -1e30


def _moe_kernel(x_ref, wg_ref, bg_ref, w1_ref, b1_ref, w2_ref, b2_ref,
                out_ref, w_scr, h_scr):
    e = pl.program_id(0)

    @pl.when(e == 0)
    def _gating():
        logits = jnp.dot(x_ref[...], wg_ref[...],
                         preferred_element_type=jnp.float32) + bg_ref[...]
        logits = logits - jnp.max(logits, axis=-1, keepdims=True)
        ex = jnp.exp(logits)
        gate = ex / jnp.sum(ex, axis=-1, keepdims=True)
        idx = jax.lax.broadcasted_iota(jnp.int32, gate.shape, 1)
        m1 = jnp.max(gate, axis=-1, keepdims=True)
        i1 = jnp.min(jnp.where(gate == m1, idx, E), axis=-1, keepdims=True)
        masked = jnp.where(idx == i1, NEG_INF, gate)
        m2 = jnp.max(masked, axis=-1, keepdims=True)
        i2 = jnp.min(jnp.where(masked == m2, idx, E), axis=-1, keepdims=True)
        e1 = jnp.exp(m1 - m1)
        e2 = jnp.exp(m2 - m1)
        w1 = e1 / (e1 + e2)
        w2 = e2 / (e1 + e2)
        w_scr[...] = jnp.where(idx == i1, w1,
                               jnp.where(idx == i2, w2, 0.0))
        out_ref[...] = jnp.zeros_like(out_ref)

    idx = jax.lax.broadcasted_iota(jnp.int32, (N_TOK, E), 1)
    w_col = jnp.sum(jnp.where(idx == e, w_scr[...], 0.0), axis=1,
                    keepdims=True)
    h = jnp.dot(x_ref[...], w1_ref[0], preferred_element_type=jnp.float32)
    h = jnp.maximum(h + b1_ref[0], 0.0)
    h_scr[...] = h
    y = jnp.dot(h_scr[...], w2_ref[0], preferred_element_type=jnp.float32)
    out_ref[...] += w_col * (y + b2_ref[0])


@functools.partial(jax.jit, static_argnums=())
def kernel(x, Wg, bg, W1, b1, W2, b2):
    grid = (E,)
    out = pl.pallas_call(
        _moe_kernel,
        grid=grid,
        in_specs=[
            pl.BlockSpec((N_TOK, D), lambda e: (0, 0)),
            pl.BlockSpec((D, E), lambda e: (0, 0)),
            pl.BlockSpec((1, E), lambda e: (0, 0)),
            pl.BlockSpec((1, D, D), lambda e: (e, 0, 0)),
            pl.BlockSpec((1, 1, D), lambda e: (e, 0, 0)),
            pl.BlockSpec((1, D, D), lambda e: (e, 0, 0)),
            pl.BlockSpec((1, 1, D), lambda e: (e, 0, 0)),
        ],
        out_specs=pl.BlockSpec((N_TOK, D), lambda e: (0, 0)),
        out_shape=jax.ShapeDtypeStruct((N_TOK, D), jnp.float32),
        scratch_shapes=[
            pltpu.VMEM((N_TOK, E), jnp.float32),
            pltpu.VMEM((N_TOK, D), jnp.float32),
        ],
        compiler_params=pltpu.CompilerParams(
            dimension_semantics=("arbitrary",),
        ),
    )(x, Wg, bg.reshape(1, E), W1, b1.reshape(E, 1, D), W2, b2.reshape(E, 1, D))
    return out
